# baseline (speedup 1.0000x reference)
import functools

import jax
import jax.numpy as jnp
from jax import lax
from jax.experimental import pallas as pl
from jax.experimental.pallas import tpu as pltpu
from jax.experimental.pallas import tpu_sc as plsc

L = 16
NC = 2
NS = 16
NW = NC * NS

_CES = [(0, 1), (2, 3), (4, 5), (6, 7),
        (0, 2), (1, 3), (4, 6), (5, 7),
        (1, 2), (5, 6),
        (0, 4), (1, 5), (2, 6), (3, 7),
        (2, 4), (3, 5),
        (1, 2), (3, 4), (5, 6)]


def _make_kernel(B, I, A, T, D, BC):
    NF = I // NW
    NG = BC // L
    NCH = B // BC
    HD = D // 2
    TS = HD + 1
    XS = NF * A + 1
    OS = NF * D + 1
    NTC = 2
    NIB = I // 128
    NDB = D // 8
    mesh = plsc.VectorSubcoreMesh(core_axis_name="c", subcore_axis_name="s",
                                  num_cores=NC, num_subcores=NS)

    @functools.partial(
        pl.kernel,
        out_type=jax.ShapeDtypeStruct((B, NDB, NIB, 8, 128), jnp.float32),
        mesh=mesh,
        scratch_types=[
            pltpu.VMEM((NTC * T * D,), jnp.float32),
            pltpu.VMEM((NF * T * TS,), jnp.int32),
            [pltpu.VMEM((BC, A, NF), jnp.float32)
             for _ in range(2)],
            pltpu.VMEM((BC * XS,), jnp.float32),
            pltpu.VMEM((BC * OS,), jnp.float32),
            [pltpu.VMEM((BC, NDB, 8, NF), jnp.float32)
             for _ in range(2)],
            [pltpu.SemaphoreType.DMA for _ in range(2)],
            [pltpu.SemaphoreType.DMA for _ in range(2)],
        ],
        compiler_params=pltpu.CompilerParams(
            needs_layout_passes=False, use_tc_tiling_on_sc=False),
    )
    def k(x4_hbm, pflat_hbm, y5_hbm, traw, tb, stx, xf, of, sty, sxs, sys):
        wid = lax.axis_index("s") * NC + lax.axis_index("c")
        f0 = wid * NF
        ib = wid // (128 // NF)
        ilo = (wid % (128 // NF)) * NF
        lane = jnp.arange(L, dtype=jnp.int32)
        rowsel = lane >> 3
        within = lane & 7
        pay = [jnp.full((L,), 1 << j, jnp.int32) for j in range(A)]
        j0_off = (255 ^ 15) * TS

        def start_x(ci):
            return pltpu.async_copy(
                x4_hbm.at[pl.ds(ci * BC, BC), ib, :, pl.ds(ilo, NF)],
                stx[ci % 2], sxs[ci % 2])

        x_descs = [None, None]
        x_descs[0] = start_x(0)

        for tc in range(NF // NTC):
            pltpu.sync_copy(
                pflat_hbm.at[pl.ds((f0 + tc * NTC) * (T * D), NTC * T * D)],
                traw)

            @plsc.parallel_loop(0, NTC * T // 2, 1, unroll=2)
            def _srow(r2):
                idx_e = r2 * (2 * D) + rowsel * D + within * 2
                ve = plsc.bitcast(plsc.load_gather(traw, [idx_e]), jnp.int32)
                vo = plsc.bitcast(plsc.load_gather(traw, [idx_e + 1]),
                                  jnp.int32)
                be = lax.shift_right_logical(
                    ve + 32767 + (lax.shift_right_logical(ve, 16) & 1), 16)
                bo = lax.shift_right_logical(
                    vo + 32767 + (lax.shift_right_logical(vo, 16) & 1), 16)
                w = (bo << 16) | be
                ri = r2 * 2 + rowsel
                m = ri & (T - 1)
                fc = ri >> 8
                s = m ^ (m >> 4)
                dst = ((tc * NTC + fc) * T + s) * TS + within
                plsc.store_scatter(tb, [dst], w)

        out_descs = [None, None]
        for ci in range(NCH):
            b0c = ci * BC
            par = ci % 2
            x_descs[par].wait()
            if ci + 1 < NCH:
                x_descs[1 - par] = start_x(ci + 1)

            stxc = stx[par]

            @plsc.parallel_loop(0, BC * (NF * A // L), 1, unroll=4)
            def _xrep(r):
                b = r >> 2
                kk = (r & 3) * L
                v = plsc.load_gather(
                    stxc, [jnp.full((L,), b, jnp.int32),
                           (kk + lane) >> 3, within])
                plsc.store_scatter(xf, [b * XS + kk + lane], v)

            @plsc.parallel_loop(0, NF * NG, 1, unroll=1)
            def _group(t):
                fl = t // NG
                g = t - fl * NG
                tbase = fl * (T * TS)
                bloc = g * L + lane
                ox = bloc * XS + fl
                v = [plsc.load_gather(xf, [ox + j * NF]) for j in range(A)]
                p = list(pay)
                for a, b in _CES:
                    c = v[a] <= v[b]
                    va, vb = v[a], v[b]
                    v[a] = jnp.where(c, va, vb)
                    v[b] = jnp.where(c, vb, va)
                    pa, pb = p[a], p[b]
                    p[a] = jnp.where(c, pa, pb)
                    p[b] = jnp.where(c, pb, pa)
                coef = [v[0]] + [v[j] - v[j - 1] for j in range(1, A)]
                m = [None] * A
                m[A - 1] = p[A - 1]
                for j in range(A - 2, 0, -1):
                    m[j] = m[j + 1] + p[j]
                base = [None] * A
                base[0] = jnp.full((L,), 0, jnp.int32) + (tbase + j0_off)
                for j in range(1, A):
                    s = m[j] ^ (m[j] >> 4)
                    base[j] = tbase + s * TS
                himask = jnp.full((L,), -65536, jnp.int32)
                oo = bloc * OS + fl
                acc_lo = [None] * HD
                acc_hi = [None] * HD
                for j in range(A):
                    for dp in range(HD):
                        w = plsc.load_gather(tb, [base[j] + dp])
                        plo = coef[j] * plsc.bitcast(w << 16, jnp.float32)
                        phi = coef[j] * plsc.bitcast(w & himask, jnp.float32)
                        if j == 0:
                            acc_lo[dp] = plo
                            acc_hi[dp] = phi
                        else:
                            acc_lo[dp] = acc_lo[dp] + plo
                            acc_hi[dp] = acc_hi[dp] + phi
                for dp in range(HD):
                    d_lo, d_hi = 2 * dp, 2 * dp + 1
                    plsc.store_scatter(
                        of, [oo + ((d_lo >> 3) * 64 + (d_lo & 7) * 8)],
                        acc_lo[dp])
                    plsc.store_scatter(
                        of, [oo + ((d_hi >> 3) * 64 + (d_hi & 7) * 8)],
                        acc_hi[dp])

            if out_descs[par] is not None:
                out_descs[par].wait()
            styc = sty[par]

            @plsc.parallel_loop(0, BC * (NF * D // L), 1, unroll=4)
            def _yrep(r):
                b = r >> 3
                k = r & 7
                kk = k * L
                v = plsc.load_gather(of, [b * OS + kk + lane])
                plsc.store_scatter(
                    styc, [jnp.full((L,), b, jnp.int32),
                           jnp.full((L,), k >> 2, jnp.int32),
                           ((kk + lane) >> 3) & 7, within], v)

            out_descs[par] = pltpu.async_copy(
                styc, y5_hbm.at[pl.ds(b0c, BC), :, ib, :, pl.ds(ilo, NF)],
                sys[par])

        for par in range(2):
            if out_descs[par] is not None:
                out_descs[par].wait()

    return k


def kernel(X, params):
    B, I, A = X.shape
    _, T, D = params.shape
    NIB = I // 128
    k = _make_kernel(B, I, A, T, D, BC=128)
    x4 = X.transpose(0, 2, 1).reshape(B, A, NIB, 128).transpose(0, 2, 1, 3)
    y5 = k(x4, params.reshape(-1))
    return y5.transpose(0, 2, 4, 1, 3).reshape(B, I, D)
